# Initial kernel scaffold; baseline (speedup 1.0000x reference)
import jax
import jax.numpy as jnp
from jax.experimental import pallas as pl


def kernel(embedded, centers):
    raise NotImplementedError("write your pallas kernel here")


# speedup vs baseline: 3.6126x; 3.6126x over previous
import functools

import jax
import jax.numpy as jnp
from jax.experimental import pallas as pl


def _dcn_block(e_ref, c_ref, lab_ref, loss_ref, *, n_total):
    i = pl.program_id(0)
    e = e_ref[...]
    c = c_ref[...]
    k = c.shape[0]

    xct = jax.lax.dot_general(
        c.astype(jnp.bfloat16),
        e.astype(jnp.bfloat16),
        (((1,), (1,)), ((), ())),
        preferred_element_type=jnp.float32,
    )
    c2 = jnp.sum(c * c, axis=1, keepdims=True)
    part = c2 - 2.0 * xct

    minv = jnp.min(part, axis=0, keepdims=True)
    iota = jax.lax.broadcasted_iota(jnp.int32, part.shape, 0)
    lab = jnp.min(jnp.where(part == minv, iota, k), axis=0, keepdims=True)
    lab_ref[0] = lab.astype(jnp.int32)

    @pl.when(i == 0)
    def _():
        loss_ref[...] = jnp.zeros((1, 1), jnp.float32)

    blk = jnp.sum(e * e) + jnp.sum(minv)
    loss_ref[...] += (blk * (1.0 / n_total)).reshape(1, 1)


def kernel(embedded, centers):
    n, d = embedded.shape
    k = centers.shape[0]
    bn = 1024
    g = n // bn

    lab3, loss = pl.pallas_call(
        functools.partial(_dcn_block, n_total=n),
        grid=(g,),
        in_specs=[
            pl.BlockSpec((bn, d), lambda i: (i, 0)),
            pl.BlockSpec((k, d), lambda i: (0, 0)),
        ],
        out_specs=[
            pl.BlockSpec((1, 1, bn), lambda i: (i, 0, 0)),
            pl.BlockSpec((1, 1), lambda i: (0, 0)),
        ],
        out_shape=[
            jax.ShapeDtypeStruct((g, 1, bn), jnp.int32),
            jax.ShapeDtypeStruct((1, 1), jnp.float32),
        ],
    )(embedded, centers)

    return lab3.reshape(n), loss[0, 0]


# speedup vs baseline: 3.9964x; 1.1062x over previous
import functools

import jax
import jax.numpy as jnp
from jax.experimental import pallas as pl
from jax.experimental.pallas import tpu as pltpu


def _dcn_block(e_ref, c_ref, lab_ref, loss_ref, *, n_total):
    e = e_ref[...]
    c = c_ref[...]
    k = c.shape[0]

    xct = jax.lax.dot_general(
        c.astype(jnp.bfloat16),
        e.astype(jnp.bfloat16),
        (((1,), (1,)), ((), ())),
        preferred_element_type=jnp.float32,
    )
    c2 = jnp.sum(c * c, axis=1, keepdims=True)
    part = c2 - 2.0 * xct

    minv = jnp.min(part, axis=0, keepdims=True)
    iota = jax.lax.broadcasted_iota(jnp.int32, part.shape, 0)
    lab = jnp.min(jnp.where(part == minv, iota, k), axis=0, keepdims=True)
    lab_ref[0] = lab.astype(jnp.int32)

    blk = jnp.sum(e * e) + jnp.sum(minv)
    loss_ref[...] = (blk * (1.0 / n_total)).reshape(1, 1, 1)


def kernel(embedded, centers):
    n, d = embedded.shape
    k = centers.shape[0]
    bn = 1024
    g = n // bn

    lab3, loss_parts = pl.pallas_call(
        functools.partial(_dcn_block, n_total=n),
        grid=(g,),
        in_specs=[
            pl.BlockSpec((bn, d), lambda i: (i, 0)),
            pl.BlockSpec((k, d), lambda i: (0, 0)),
        ],
        out_specs=[
            pl.BlockSpec((1, 1, bn), lambda i: (i, 0, 0)),
            pl.BlockSpec((1, 1, 1), lambda i: (i, 0, 0)),
        ],
        out_shape=[
            jax.ShapeDtypeStruct((g, 1, bn), jnp.int32),
            jax.ShapeDtypeStruct((g, 1, 1), jnp.float32),
        ],
        compiler_params=pltpu.CompilerParams(
            dimension_semantics=("parallel",),
        ),
    )(embedded, centers)

    return lab3.reshape(n), jnp.sum(loss_parts)
